# Initial kernel scaffold; baseline (speedup 1.0000x reference)
import jax
import jax.numpy as jnp
from jax.experimental import pallas as pl


def kernel(x, edge_index, W1, b1, W2, b2, W_ih_f, W_hh_f, b_ih_f, b_hh_f, W_ih_b, W_hh_b, b_ih_b, b_hh_b, W_att, b_att, W_lin, b_lin):
    raise NotImplementedError("write your pallas kernel here")


# speedup vs baseline: 19.9970x; 19.9970x over previous
import functools

import jax
import jax.numpy as jnp
from jax import lax
from jax.experimental import pallas as pl
from jax.experimental.pallas import tpu as pltpu
from jax.experimental.pallas import tpu_sc as plsc

N = 10000
NP = 10112
H = 16
NC = 2
NS = 16
CHUNK = 128
CPW = 80
EP = NC * NS * CPW * CHUNK
RPS = NP // NS

_f32 = jnp.float32


def _sc_body(table, srcp, dstp, out, sidx, didx, rows, tmp, shared, sem):
    c = lax.axis_index("c")
    s = lax.axis_index("s")
    wid = c * NS + s
    r0 = s * RPS
    pltpu.sync_copy(table.at[pl.ds(r0, RPS)], tmp)
    pltpu.sync_copy(tmp, shared.at[pl.ds(r0, RPS)])
    pltpu.sync_copy(srcp.at[pl.ds(wid * CPW, CPW)], sidx)
    pltpu.sync_copy(dstp.at[pl.ds(wid * CPW, CPW)], didx)
    plsc.subcore_barrier()

    def body(j, carry):
        pltpu.async_copy(table.at[sidx.at[j]], rows, sem).wait()
        pltpu.sync_copy(rows, shared.at[didx.at[j]], add=True)
        return carry

    lax.fori_loop(0, CPW, body, 0)
    plsc.subcore_barrier()
    pltpu.sync_copy(shared.at[pl.ds(r0, RPS)], tmp)
    pltpu.sync_copy(tmp, out.at[c, pl.ds(r0, RPS)])


_sc_prop = functools.partial(
    pl.kernel,
    out_type=jax.ShapeDtypeStruct((NC, NP, H), _f32),
    mesh=plsc.VectorSubcoreMesh(core_axis_name="c", subcore_axis_name="s"),
    scratch_types=[
        pltpu.VMEM((CPW, CHUNK), jnp.int32),
        pltpu.VMEM((CPW, CHUNK), jnp.int32),
        pltpu.VMEM((CHUNK, H), _f32),
        pltpu.VMEM((RPS, H), _f32),
        pltpu.VMEM_SHARED((NP, H), _f32),
        pltpu.SemaphoreType.DMA,
    ],
    compiler_params=pltpu.CompilerParams(use_tc_tiling_on_sc=False),
)(_sc_body)


def _tc_a_body(x, w1, d0, d1, dinv, ys1):
    dv = lax.rsqrt(d0[...] + d1[...] - 1.0)
    dinv[...] = dv
    ys1[...] = dv * jnp.dot(x[...], w1[...], preferred_element_type=_f32)


def _tc_d_body(p0, p1, ys1, dinv, b1, w2, x1, ys2):
    dv = dinv[...]
    xx = jnp.maximum(dv * (p0[...] + p1[...] - ys1[...]) + b1[...], 0.0)
    x1[...] = xx
    ys2[...] = dv * jnp.dot(xx, w2[...], preferred_element_type=_f32)


def _lstm_cell(x, h, c, wih, whh, b):
    g = jnp.dot(x, wih, preferred_element_type=_f32) + b
    if h is not None:
        g = g + jnp.dot(h, whh, preferred_element_type=_f32)
    i = jax.nn.sigmoid(g[:, 0:32])
    f = jax.nn.sigmoid(g[:, 32:64])
    gg = jnp.tanh(g[:, 64:96])
    o = jax.nn.sigmoid(g[:, 96:128])
    cn = i * gg if c is None else f * c + i * gg
    return o * jnp.tanh(cn), cn


def _tc_f_body(p0, p1, ys2, dinv, b2, x1r, wihf, whhf, bff, wihb, whhb, bbb,
               watt, ys3):
    dv = dinv[...]
    x1 = x1r[...]
    x2 = jnp.maximum(dv * (p0[...] + p1[...] - ys2[...]) + b2[...], 0.0)
    hf1, cf1 = _lstm_cell(x1, None, None, wihf[...], whhf[...], bff[...])
    hf2, _ = _lstm_cell(x2, hf1, cf1, wihf[...], whhf[...], bff[...])
    hb1, cb1 = _lstm_cell(x2, None, None, wihb[...], whhb[...], bbb[...])
    hb2, _ = _lstm_cell(x1, hb1, cb1, wihb[...], whhb[...], bbb[...])
    w = watt[...]
    a0 = jnp.sum(jnp.concatenate([hf1, hb2], 1) * w, 1, keepdims=True)
    a1 = jnp.sum(jnp.concatenate([hf2, hb1], 1) * w, 1, keepdims=True)
    a = jax.nn.sigmoid(a0 - a1)
    ys3[...] = dv * (a * x1 + (1.0 - a) * x2)


def _tc_h_body(p0, p1, ys3, dinv, wlin, blin, out):
    xp = dinv[...] * (p0[...] + p1[...] - ys3[...])
    z = jnp.dot(xp, wlin[...], preferred_element_type=_f32) + blin[...]
    m = jnp.max(z, 1, keepdims=True)
    out[...] = z - m - jnp.log(jnp.sum(jnp.exp(z - m), 1, keepdims=True))


_R = 2000


def _row_spec(width):
    return pl.BlockSpec((_R, width), lambda i: (i, 0))


def _full_spec(shape):
    return pl.BlockSpec(shape, lambda i: (0,) * len(shape))


def _call(body, in_widths, n_out):
    in_specs = [_row_spec(w) if isinstance(w, int) else _full_spec(w)
                for w in in_widths]
    return pl.pallas_call(
        body,
        grid=(N // _R,),
        in_specs=in_specs,
        out_specs=[_row_spec(H)] * n_out,
        out_shape=[jax.ShapeDtypeStruct((N, H), _f32) for _ in range(n_out)],
    )


def _pad_rows(y):
    return jnp.concatenate([y, jnp.zeros((NP - N, H), _f32)], axis=0)


def kernel(x, edge_index, W1, b1, W2, b2, W_ih_f, W_hh_f, b_ih_f, b_hh_f,
           W_ih_b, W_hh_b, b_ih_b, b_hh_b, W_att, b_att, W_lin, b_lin):
    E = edge_index.shape[1]
    src = jnp.concatenate(
        [edge_index[0], jnp.zeros((EP - E,), jnp.int32)]).reshape(-1, CHUNK)
    dst = jnp.concatenate(
        [edge_index[1], jnp.full((EP - E,), N, jnp.int32)]).reshape(-1, CHUNK)

    ones_t = jnp.ones((NP, H), _f32)
    degp = _sc_prop(ones_t, src, dst)
    dinv, ys1 = _call(_tc_a_body, [128, (128, 16), 16, 16], 2)(
        x, W1, degp[0, :N], degp[1, :N])

    p = _sc_prop(_pad_rows(ys1), src, dst)
    x1, ys2 = _call(_tc_d_body, [16, 16, 16, 16, (1, 16), (16, 16)], 2)(
        p[0, :N], p[1, :N], ys1, dinv, b1.reshape(1, H), W2)

    p = _sc_prop(_pad_rows(ys2), src, dst)
    (ys3,) = _call(_tc_f_body,
                   [16, 16, 16, 16, (1, 16), 16, (16, 128), (32, 128),
                    (1, 128), (16, 128), (32, 128), (1, 128), (1, 64)], 1)(
        p[0, :N], p[1, :N], ys2, dinv, b2.reshape(1, H), x1,
        W_ih_f.T, W_hh_f.T, (b_ih_f + b_hh_f).reshape(1, -1),
        W_ih_b.T, W_hh_b.T, (b_ih_b + b_hh_b).reshape(1, -1), W_att)

    p = _sc_prop(_pad_rows(ys3), src, dst)
    (out,) = _call(_tc_h_body, [16, 16, 16, 16, (16, 16), (1, 16)], 1)(
        p[0, :N], p[1, :N], ys3, dinv, W_lin, b_lin.reshape(1, -1))
    return out
